# baseline (speedup 1.0000x reference)
import jax
import jax.numpy as jnp
from jax import lax
from jax.experimental import pallas as pl
from jax.experimental.pallas import tpu as pltpu
from jax.experimental.pallas import tpu_sc as plsc

BATCH = 16384
D = 128
L = 16
NC = 2
NS = 16
NW = NC * NS
ROWS_PER_W = BATCH // NW
CHUNK = 64
NCHUNKS = ROWS_PER_W // CHUNK


def _permute_body(tgt_hbm, perm_hbm, out_hbm, perm_v,
                  in0, in1, out0, out1,
                  sem_in0, sem_in1, sem_out0, sem_out1):
    wid = lax.axis_index("s") * NC + lax.axis_index("c")
    row0 = wid * ROWS_PER_W

    in_bufs = (in0, in1)
    out_bufs = (out0, out1)
    sem_in = (sem_in0, sem_in1)
    sem_out = (sem_out0, sem_out1)

    def rows(c):
        return pl.ds(row0 + c * CHUNK, CHUNK)

    in_dma = [None] * NCHUNKS
    out_dma = [None] * NCHUNKS
    in_dma[0] = pltpu.async_copy(tgt_hbm.at[rows(0)], in_bufs[0], sem_in[0])
    if NCHUNKS > 1:
        in_dma[1] = pltpu.async_copy(tgt_hbm.at[rows(1)], in_bufs[1], sem_in[1])

    pltpu.sync_copy(perm_hbm, perm_v)
    pslices = [perm_v[pl.ds(j * L, L)] for j in range(D // L)]

    def compute(in_ref, out_ref):
        @plsc.parallel_loop(0, CHUNK, unroll=2)
        def _(r):
            rvec = jnp.full((L,), 0, jnp.int32) + r
            for j in range(D // L):
                out_ref[r, pl.ds(j * L, L)] = plsc.load_gather(
                    in_ref, [rvec, pslices[j]])

    for c in range(NCHUNKS):
        b = c % 2
        in_dma[c].wait()
        if c >= 2:
            out_dma[c - 2].wait()
        compute(in_bufs[b], out_bufs[b])
        if c + 2 < NCHUNKS:
            in_dma[c + 2] = pltpu.async_copy(
                tgt_hbm.at[rows(c + 2)], in_bufs[b], sem_in[b])
        out_dma[c] = pltpu.async_copy(out_bufs[b], out_hbm.at[rows(c)],
                                      sem_out[b])
    for c in range(max(0, NCHUNKS - 2), NCHUNKS):
        out_dma[c].wait()


def kernel(target, permutation):
    mesh = plsc.VectorSubcoreMesh(core_axis_name="c", subcore_axis_name="s")
    k = pl.kernel(
        _permute_body,
        out_type=jax.ShapeDtypeStruct((BATCH, D), jnp.float32),
        mesh=mesh,
        compiler_params=pltpu.CompilerParams(needs_layout_passes=False),
        scratch_types=[
            pltpu.VMEM((D,), jnp.int32),
            pltpu.VMEM((CHUNK, D), jnp.float32),
            pltpu.VMEM((CHUNK, D), jnp.float32),
            pltpu.VMEM((CHUNK, D), jnp.float32),
            pltpu.VMEM((CHUNK, D), jnp.float32),
            pltpu.SemaphoreType.DMA,
            pltpu.SemaphoreType.DMA,
            pltpu.SemaphoreType.DMA,
            pltpu.SemaphoreType.DMA,
        ],
    )
    return k(target, permutation)


# speedup vs baseline: 1.0767x; 1.0767x over previous
import jax
import jax.numpy as jnp
from jax import lax
from jax.experimental import pallas as pl
from jax.experimental.pallas import tpu as pltpu
from jax.experimental.pallas import tpu_sc as plsc

BATCH = 16384
D = 128
L = 16
NC = 2
NS = 16
NW = NC * NS
ROWS_PER_W = BATCH // NW
CHUNK = 128
NCHUNKS = ROWS_PER_W // CHUNK


def _permute_body(tgt_hbm, perm_hbm, out_hbm, perm_v,
                  in0, in1, out0, out1,
                  sem_in0, sem_in1, sem_out0, sem_out1):
    wid = lax.axis_index("s") * NC + lax.axis_index("c")
    row0 = wid * ROWS_PER_W

    in_bufs = (in0, in1)
    out_bufs = (out0, out1)
    sem_in = (sem_in0, sem_in1)
    sem_out = (sem_out0, sem_out1)

    def rows(c):
        return pl.ds(row0 + c * CHUNK, CHUNK)

    in_dma = [None] * NCHUNKS
    out_dma = [None] * NCHUNKS
    in_dma[0] = pltpu.async_copy(tgt_hbm.at[rows(0)], in_bufs[0], sem_in[0])
    if NCHUNKS > 1:
        in_dma[1] = pltpu.async_copy(tgt_hbm.at[rows(1)], in_bufs[1], sem_in[1])

    pltpu.sync_copy(perm_hbm, perm_v)
    pslices = [perm_v[pl.ds(j * L, L)] for j in range(D // L)]

    def compute(in_ref, out_ref):
        @plsc.parallel_loop(0, CHUNK, unroll=4)
        def _(r):
            rvec = jnp.full((L,), 0, jnp.int32) + r
            for j in range(D // L):
                out_ref[r, pl.ds(j * L, L)] = plsc.load_gather(
                    in_ref, [rvec, pslices[j]])

    for c in range(NCHUNKS):
        b = c % 2
        in_dma[c].wait()
        if c >= 2:
            out_dma[c - 2].wait()
        compute(in_bufs[b], out_bufs[b])
        if c + 2 < NCHUNKS:
            in_dma[c + 2] = pltpu.async_copy(
                tgt_hbm.at[rows(c + 2)], in_bufs[b], sem_in[b])
        out_dma[c] = pltpu.async_copy(out_bufs[b], out_hbm.at[rows(c)],
                                      sem_out[b])
    for c in range(max(0, NCHUNKS - 2), NCHUNKS):
        out_dma[c].wait()


def kernel(target, permutation):
    mesh = plsc.VectorSubcoreMesh(core_axis_name="c", subcore_axis_name="s")
    k = pl.kernel(
        _permute_body,
        out_type=jax.ShapeDtypeStruct((BATCH, D), jnp.float32),
        mesh=mesh,
        compiler_params=pltpu.CompilerParams(needs_layout_passes=False),
        scratch_types=[
            pltpu.VMEM((D,), jnp.int32),
            pltpu.VMEM((CHUNK, D), jnp.float32),
            pltpu.VMEM((CHUNK, D), jnp.float32),
            pltpu.VMEM((CHUNK, D), jnp.float32),
            pltpu.VMEM((CHUNK, D), jnp.float32),
            pltpu.SemaphoreType.DMA,
            pltpu.SemaphoreType.DMA,
            pltpu.SemaphoreType.DMA,
            pltpu.SemaphoreType.DMA,
        ],
    )
    return k(target, permutation)
